# Initial kernel scaffold; baseline (speedup 1.0000x reference)
import jax
import jax.numpy as jnp
from jax.experimental import pallas as pl


def kernel(x, edge_index, edge_weight, W1, b1, W2, b2):
    raise NotImplementedError("write your pallas kernel here")


# speedup vs baseline: 3.6566x; 3.6566x over previous
import functools

import jax
import jax.numpy as jnp
from jax import lax
from jax.experimental import pallas as pl
from jax.experimental.pallas import tpu as pltpu
from jax.experimental.pallas import tpu_sc as plsc

_N = 10000
_E = 160000
_NSUB = 16
_NCORE = 2
_CH = 128


def _matmul_halves(x, W, Rb):
    N, K = x.shape
    F2 = W.shape[1] // 2

    def body(x_ref, w_ref, o_ref):
        o_ref[0] = jnp.dot(x_ref[...], w_ref[...],
                           preferred_element_type=jnp.float32)

    return pl.pallas_call(
        body,
        grid=(2, N // Rb),
        in_specs=[pl.BlockSpec((Rb, K), lambda h, r: (r, 0)),
                  pl.BlockSpec((K, F2), lambda h, r: (0, h))],
        out_specs=pl.BlockSpec((1, Rb, F2), lambda h, r: (h, r, 0)),
        out_shape=jax.ShapeDtypeStruct((2, N, F2), jnp.float32),
    )(x, W)


def _relu_matmul(agg, b, W, Rb):
    _, N, K2 = agg.shape
    F = W.shape[1]

    def body(a_ref, b_ref, w_ref, o_ref):
        h0 = jnp.maximum(a_ref[0] + b_ref[0], 0.0)
        h1 = jnp.maximum(a_ref[1] + b_ref[1], 0.0)
        w = w_ref[...]
        o_ref[...] = (jnp.dot(h0, w[:K2], preferred_element_type=jnp.float32)
                      + jnp.dot(h1, w[K2:], preferred_element_type=jnp.float32))

    return pl.pallas_call(
        body,
        grid=(N // Rb,),
        in_specs=[pl.BlockSpec((2, Rb, K2), lambda r: (0, r, 0)),
                  pl.BlockSpec((2, 1, K2), lambda r: (0, 0, 0)),
                  pl.BlockSpec((2 * K2, F), lambda r: (0, 0))],
        out_specs=pl.BlockSpec((Rb, F), lambda r: (r, 0)),
        out_shape=jax.ShapeDtypeStruct((N, F), jnp.float32),
    )(agg, b.reshape(2, 1, K2), W)


def _combine_bias(parts, b, Rb):
    _, N, F = parts.shape

    def body(a_ref, b_ref, o_ref):
        o_ref[...] = a_ref[0] + a_ref[1] + b_ref[...]

    return pl.pallas_call(
        body,
        grid=(N // Rb,),
        in_specs=[pl.BlockSpec((2, Rb, F), lambda r: (0, r, 0)),
                  pl.BlockSpec((1, F), lambda r: (0, 0))],
        out_specs=pl.BlockSpec((Rb, F), lambda r: (r, 0)),
        out_shape=jax.ShapeDtypeStruct((N, F), jnp.float32),
    )(parts, b.reshape(1, F))


def _sc_spmm(sup, src, dst, w, feature_split):
    n_sup_rows, F2 = sup.shape
    mesh = plsc.VectorSubcoreMesh(core_axis_name="c", subcore_axis_name="s")
    n_chunks = _E // _CH
    chunks_per_core = n_chunks // _NCORE
    rows_per_sub = 624
    tail_base = _NSUB * rows_per_sub
    tail_rows = _N - tail_base

    @functools.partial(
        pl.kernel,
        mesh=mesh,
        out_type=jax.ShapeDtypeStruct((2, _N, F2), jnp.float32),
        scratch_types=[
            pltpu.VMEM((_CH,), jnp.int32),
            pltpu.VMEM((_CH,), jnp.int32),
            pltpu.VMEM((_CH,), jnp.float32),
            pltpu.VMEM((_CH, F2), jnp.float32),
            pltpu.VMEM_SHARED((_N, F2), jnp.float32),
            pltpu.SemaphoreType.DMA,
        ],
    )
    def k(sup_hbm, src_hbm, dst_hbm, w_hbm, out_hbm,
          src_v, dst_v, w_v, rows_v, acc_sh, sem):
        c = lax.axis_index("c")
        s = lax.axis_index("s")

        @pl.loop(0, _CH)
        def _(r):
            @pl.loop(0, F2, step=16)
            def _(f):
                rows_v[r, pl.ds(f, 16)] = jnp.zeros((16,), jnp.float32)

        base_row = pl.multiple_of(s * rows_per_sub, 8)
        off = 0
        while off < rows_per_sub:
            sz = min(_CH, rows_per_sub - off)
            pltpu.sync_copy(rows_v.at[pl.ds(0, sz)],
                            acc_sh.at[pl.ds(base_row + off, sz)])
            off += sz

        @pl.when(s == _NSUB - 1)
        def _():
            pltpu.sync_copy(rows_v.at[pl.ds(0, tail_rows)],
                            acc_sh.at[pl.ds(tail_base, tail_rows)])

        plsc.subcore_barrier()

        if feature_split:
            chunk_lo = s
            chunk_hi = n_chunks
        else:
            chunk_lo = c * chunks_per_core + s
            chunk_hi = (c + 1) * chunks_per_core

        @pl.loop(chunk_lo, chunk_hi, step=_NSUB)
        def _(ci):
            ebase = pl.multiple_of(ci * _CH, _CH)
            pltpu.sync_copy(src_hbm.at[pl.ds(ebase, _CH)], src_v)
            pltpu.sync_copy(dst_hbm.at[pl.ds(ebase, _CH)], dst_v)
            pltpu.sync_copy(w_hbm.at[pl.ds(ebase, _CH)], w_v)
            if feature_split:
                roff = c * _N

                @pl.loop(0, _CH, step=16)
                def _(i):
                    src_v[pl.ds(i, 16)] = src_v[pl.ds(i, 16)] + roff

            pltpu.async_copy(sup_hbm.at[src_v], rows_v, sem).wait()

            @pl.loop(0, _CH, step=16)
            def _(g):
                wvec = w_v[pl.ds(g, 16)]
                for j in range(16):
                    wj = wvec.at[jnp.full((16,), j, jnp.int32)].get(
                        mode="promise_in_bounds")

                    @pl.loop(0, F2, step=16)
                    def _(f):
                        rows_v[g + j, pl.ds(f, 16)] = (
                            rows_v[g + j, pl.ds(f, 16)] * wj)

            pltpu.sync_copy(rows_v, acc_sh.at[dst_v], add=True)

        plsc.subcore_barrier()
        pltpu.sync_copy(acc_sh.at[pl.ds(base_row, rows_per_sub)],
                        out_hbm.at[c].at[pl.ds(base_row, rows_per_sub)])

        @pl.when(s == _NSUB - 1)
        def _():
            pltpu.sync_copy(acc_sh.at[pl.ds(tail_base, tail_rows)],
                            out_hbm.at[c].at[pl.ds(tail_base, tail_rows)])

    return k(sup, src, dst, w)


def kernel(x, edge_index, edge_weight, W1, b1, W2, b2):
    src = edge_index[0]
    dst = edge_index[1]
    sup1 = _matmul_halves(x, W1, 1000)
    agg1 = _sc_spmm(sup1.reshape(2 * _N, 128), src, dst, edge_weight,
                    feature_split=True)
    sup2 = _relu_matmul(agg1, b1, W2, 1000)
    agg2 = _sc_spmm(sup2, src, dst, edge_weight,
                    feature_split=False)
    logits = _combine_bias(agg2, b2, 1000)
    return (logits, logits)


# speedup vs baseline: 4.9818x; 1.3624x over previous
import functools

import jax
import jax.numpy as jnp
from jax import lax
from jax.experimental import pallas as pl
from jax.experimental.pallas import tpu as pltpu
from jax.experimental.pallas import tpu_sc as plsc

_N = 10000
_E = 160000
_NSUB = 16
_NCORE = 2
_CH = 128


def _matmul_halves(x, W, Rb):
    N, K = x.shape
    F2 = W.shape[1] // 2

    def body(x_ref, w_ref, o_ref):
        o_ref[0] = jnp.dot(x_ref[...], w_ref[...],
                           preferred_element_type=jnp.float32)

    return pl.pallas_call(
        body,
        grid=(2, N // Rb),
        in_specs=[pl.BlockSpec((Rb, K), lambda h, r: (r, 0)),
                  pl.BlockSpec((K, F2), lambda h, r: (0, h))],
        out_specs=pl.BlockSpec((1, Rb, F2), lambda h, r: (h, r, 0)),
        out_shape=jax.ShapeDtypeStruct((2, N, F2), jnp.float32),
    )(x, W)


def _relu_matmul(agg, b, W, Rb):
    _, N, K2 = agg.shape
    F = W.shape[1]

    def body(a_ref, b_ref, w_ref, o_ref):
        h0 = jnp.maximum(a_ref[0] + b_ref[0], 0.0)
        h1 = jnp.maximum(a_ref[1] + b_ref[1], 0.0)
        w = w_ref[...]
        o_ref[...] = (jnp.dot(h0, w[:K2], preferred_element_type=jnp.float32)
                      + jnp.dot(h1, w[K2:], preferred_element_type=jnp.float32))

    return pl.pallas_call(
        body,
        grid=(N // Rb,),
        in_specs=[pl.BlockSpec((2, Rb, K2), lambda r: (0, r, 0)),
                  pl.BlockSpec((2, 1, K2), lambda r: (0, 0, 0)),
                  pl.BlockSpec((2 * K2, F), lambda r: (0, 0))],
        out_specs=pl.BlockSpec((Rb, F), lambda r: (r, 0)),
        out_shape=jax.ShapeDtypeStruct((N, F), jnp.float32),
    )(agg, b.reshape(2, 1, K2), W)


def _combine_bias(parts, b, Rb):
    _, N, F = parts.shape

    def body(a_ref, b_ref, o_ref):
        o_ref[...] = a_ref[0] + a_ref[1] + b_ref[...]

    return pl.pallas_call(
        body,
        grid=(N // Rb,),
        in_specs=[pl.BlockSpec((2, Rb, F), lambda r: (0, r, 0)),
                  pl.BlockSpec((1, F), lambda r: (0, 0))],
        out_specs=pl.BlockSpec((Rb, F), lambda r: (r, 0)),
        out_shape=jax.ShapeDtypeStruct((N, F), jnp.float32),
    )(parts, b.reshape(1, F))


def _sc_spmm(sup, src, dst, w, feature_split):
    n_sup_rows, F2 = sup.shape
    mesh = plsc.VectorSubcoreMesh(core_axis_name="c", subcore_axis_name="s")
    n_chunks = _E // _CH
    chunks_per_core = n_chunks // _NCORE
    rows_per_sub = 624
    tail_base = _NSUB * rows_per_sub
    tail_rows = _N - tail_base

    total = n_chunks if feature_split else chunks_per_core
    max_slots = (total + _NSUB - 1) // _NSUB
    pairs = (max_slots + 1) // 2

    @functools.partial(
        pl.kernel,
        mesh=mesh,
        out_type=jax.ShapeDtypeStruct((2, _N, F2), jnp.float32),
        scratch_types=[
            pltpu.VMEM((_CH,), jnp.int32),
            pltpu.VMEM((_CH,), jnp.int32),
            pltpu.VMEM((_CH,), jnp.float32),
            pltpu.VMEM((_CH, F2), jnp.float32),
            pltpu.VMEM((_CH,), jnp.int32),
            pltpu.VMEM((_CH,), jnp.int32),
            pltpu.VMEM((_CH,), jnp.float32),
            pltpu.VMEM((_CH, F2), jnp.float32),
            pltpu.VMEM_SHARED((_N, F2), jnp.float32),
            pltpu.SemaphoreType.DMA,
            pltpu.SemaphoreType.DMA,
        ],
    )
    def k(sup_hbm, src_hbm, dst_hbm, w_hbm, out_hbm,
          src_a, dst_a, w_a, rows_a, src_b, dst_b, w_b, rows_b,
          acc_sh, sem_a, sem_b):
        c = lax.axis_index("c")
        s = lax.axis_index("s")
        sets = ((src_a, dst_a, w_a, rows_a, sem_a),
                (src_b, dst_b, w_b, rows_b, sem_b))
        rows_v = rows_a

        @pl.loop(0, _CH)
        def _(r):
            @pl.loop(0, F2, step=16)
            def _(f):
                rows_v[r, pl.ds(f, 16)] = jnp.zeros((16,), jnp.float32)

        base_row = pl.multiple_of(s * rows_per_sub, 8)
        off = 0
        while off < rows_per_sub:
            sz = min(_CH, rows_per_sub - off)
            pltpu.sync_copy(rows_v.at[pl.ds(0, sz)],
                            acc_sh.at[pl.ds(base_row + off, sz)])
            off += sz

        @pl.when(s == _NSUB - 1)
        def _():
            pltpu.sync_copy(rows_v.at[pl.ds(0, tail_rows)],
                            acc_sh.at[pl.ds(tail_base, tail_rows)])

        plsc.subcore_barrier()

        chunk0 = 0 if feature_split else c * chunks_per_core

        def in_range(slot):
            return s + slot * _NSUB < total

        def issue(slot, st):
            src_v, dst_v, w_v, rows_v, sem = st
            ci = chunk0 + s + slot * _NSUB
            ebase = pl.multiple_of(ci * _CH, _CH)
            pltpu.sync_copy(src_hbm.at[pl.ds(ebase, _CH)], src_v)
            pltpu.sync_copy(dst_hbm.at[pl.ds(ebase, _CH)], dst_v)
            pltpu.sync_copy(w_hbm.at[pl.ds(ebase, _CH)], w_v)
            if feature_split:
                roff = c * _N

                @pl.loop(0, _CH, step=16)
                def _(i):
                    src_v[pl.ds(i, 16)] = src_v[pl.ds(i, 16)] + roff

            pltpu.make_async_copy(sup_hbm.at[src_v], rows_v, sem).start()

        def process(st):
            src_v, dst_v, w_v, rows_v, sem = st
            pltpu.make_async_copy(sup_hbm.at[src_v], rows_v, sem).wait()

            @pl.loop(0, _CH, step=16)
            def _(g):
                wvec = w_v[pl.ds(g, 16)]
                for j in range(16):
                    wj = wvec.at[jnp.full((16,), j, jnp.int32)].get(
                        mode="promise_in_bounds")

                    @pl.loop(0, F2, step=16)
                    def _(f):
                        rows_v[g + j, pl.ds(f, 16)] = (
                            rows_v[g + j, pl.ds(f, 16)] * wj)

            pltpu.sync_copy(rows_v, acc_sh.at[dst_v], add=True)

        @pl.when(in_range(0))
        def _():
            issue(0, sets[0])

        @pl.loop(0, pairs)
        def _(kk):
            for p in (0, 1):
                slot = 2 * kk + p

                @pl.when(in_range(slot + 1))
                def _():
                    issue(slot + 1, sets[1 - p])

                @pl.when(in_range(slot))
                def _():
                    process(sets[p])

        plsc.subcore_barrier()
        pltpu.sync_copy(acc_sh.at[pl.ds(base_row, rows_per_sub)],
                        out_hbm.at[c].at[pl.ds(base_row, rows_per_sub)])

        @pl.when(s == _NSUB - 1)
        def _():
            pltpu.sync_copy(acc_sh.at[pl.ds(tail_base, tail_rows)],
                            out_hbm.at[c].at[pl.ds(tail_base, tail_rows)])

    return k(sup, src, dst, w)


def kernel(x, edge_index, edge_weight, W1, b1, W2, b2):
    src = edge_index[0]
    dst = edge_index[1]
    sup1 = _matmul_halves(x, W1, 1000)
    agg1 = _sc_spmm(sup1.reshape(2 * _N, 128), src, dst, edge_weight,
                    feature_split=True)
    sup2 = _relu_matmul(agg1, b1, W2, 1000)
    agg2 = _sc_spmm(sup2, src, dst, edge_weight,
                    feature_split=False)
    logits = _combine_bias(agg2, b2, 1000)
    return (logits, logits)


# speedup vs baseline: 6.5634x; 1.3175x over previous
import dataclasses
import functools

import jax
import jax.numpy as jnp
from jax import lax
from jax.experimental import pallas as pl
from jax.experimental.pallas import tpu as pltpu
from jax.experimental.pallas import tpu_sc as plsc

_N = 10000
_E = 160000
_NSUB = 16
_NCORE = 2
_CH = 128


def _matmul_halves(x, W, Rb):
    N, K = x.shape
    F2 = W.shape[1] // 2

    def body(x_ref, w_ref, o_ref):
        o_ref[0] = jnp.dot(x_ref[...], w_ref[...],
                           preferred_element_type=jnp.float32)

    return pl.pallas_call(
        body,
        grid=(2, N // Rb),
        in_specs=[pl.BlockSpec((Rb, K), lambda h, r: (r, 0)),
                  pl.BlockSpec((K, F2), lambda h, r: (0, h))],
        out_specs=pl.BlockSpec((1, Rb, F2), lambda h, r: (h, r, 0)),
        out_shape=jax.ShapeDtypeStruct((2, N, F2), jnp.float32),
    )(x, W)


def _relu_matmul(agg, b, W, Rb):
    _, N, K2 = agg.shape
    F = W.shape[1]

    def body(a_ref, b_ref, w_ref, o_ref):
        h0 = jnp.maximum(a_ref[0] + b_ref[0], 0.0)
        h1 = jnp.maximum(a_ref[1] + b_ref[1], 0.0)
        w = w_ref[...]
        o_ref[...] = (jnp.dot(h0, w[:K2], preferred_element_type=jnp.float32)
                      + jnp.dot(h1, w[K2:], preferred_element_type=jnp.float32))

    return pl.pallas_call(
        body,
        grid=(N // Rb,),
        in_specs=[pl.BlockSpec((2, Rb, K2), lambda r: (0, r, 0)),
                  pl.BlockSpec((2, 1, K2), lambda r: (0, 0, 0)),
                  pl.BlockSpec((2 * K2, F), lambda r: (0, 0))],
        out_specs=pl.BlockSpec((Rb, F), lambda r: (r, 0)),
        out_shape=jax.ShapeDtypeStruct((N, F), jnp.float32),
    )(agg, b.reshape(2, 1, K2), W)


def _combine_bias(parts, b, Rb):
    _, N, F = parts.shape

    def body(a_ref, b_ref, o_ref):
        o_ref[...] = a_ref[0] + a_ref[1] + b_ref[...]

    return pl.pallas_call(
        body,
        grid=(N // Rb,),
        in_specs=[pl.BlockSpec((2, Rb, F), lambda r: (0, r, 0)),
                  pl.BlockSpec((1, F), lambda r: (0, 0))],
        out_specs=pl.BlockSpec((Rb, F), lambda r: (r, 0)),
        out_shape=jax.ShapeDtypeStruct((N, F), jnp.float32),
    )(parts, b.reshape(1, F))


def _sc_spmm(sup, sw, dst, feature_split):
    n_sup_rows, F2 = sup.shape
    mesh = plsc.VectorSubcoreMesh(core_axis_name="c", subcore_axis_name="s")
    n_chunks = _E // _CH
    chunks_per_core = n_chunks // _NCORE
    rows_per_sub = 624
    tail_base = _NSUB * rows_per_sub
    tail_rows = _N - tail_base

    total = n_chunks if feature_split else chunks_per_core
    max_slots = (total + _NSUB - 1) // _NSUB
    pairs = (max_slots + 1) // 2

    cp = pltpu.CompilerParams()
    if "needs_layout_passes" in pltpu.CompilerParams.__dataclass_fields__:
        cp = dataclasses.replace(cp, needs_layout_passes=False)

    @functools.partial(
        pl.kernel,
        mesh=mesh,
        compiler_params=cp,
        out_type=jax.ShapeDtypeStruct((2, _N, F2), jnp.float32),
        scratch_types=[
            pltpu.VMEM((2 * _CH,), jnp.int32),
            pltpu.VMEM((_CH,), jnp.int32),
            pltpu.VMEM((_CH, F2), jnp.float32),
            pltpu.VMEM((2 * _CH,), jnp.int32),
            pltpu.VMEM((_CH,), jnp.int32),
            pltpu.VMEM((_CH, F2), jnp.float32),
            pltpu.VMEM_SHARED((_N, F2), jnp.float32),
            pltpu.SemaphoreType.DMA,
            pltpu.SemaphoreType.DMA,
            pltpu.SemaphoreType.DMA,
            pltpu.SemaphoreType.DMA,
        ],
    )
    def k(sup_hbm, sw_hbm, dst_hbm, out_hbm,
          sw_a, dst_a, rows_a, sw_b, dst_b, rows_b,
          acc_sh, gsem_a, gsem_b, isem_a, isem_b):
        c = lax.axis_index("c")
        s = lax.axis_index("s")
        sets = ((sw_a, dst_a, rows_a, gsem_a, isem_a),
                (sw_b, dst_b, rows_b, gsem_b, isem_b))
        rows_v = rows_a

        @pl.loop(0, _CH)
        def _(r):
            @pl.loop(0, F2, step=16)
            def _(f):
                rows_v[r, pl.ds(f, 16)] = jnp.zeros((16,), jnp.float32)

        base_row = pl.multiple_of(s * rows_per_sub, 8)
        off = 0
        while off < rows_per_sub:
            sz = min(_CH, rows_per_sub - off)
            pltpu.sync_copy(rows_v.at[pl.ds(0, sz)],
                            acc_sh.at[pl.ds(base_row + off, sz)])
            off += sz

        @pl.when(s == _NSUB - 1)
        def _():
            pltpu.sync_copy(rows_v.at[pl.ds(0, tail_rows)],
                            acc_sh.at[pl.ds(tail_base, tail_rows)])

        plsc.subcore_barrier()

        chunk0 = 0 if feature_split else c * chunks_per_core

        def in_range(slot):
            return s + slot * _NSUB < total

        def idx_copies(slot, st):
            sw_v, dst_v, rows_v, gsem, isem = st
            ci = chunk0 + s + slot * _NSUB
            sbase = pl.multiple_of(ci * (2 * _CH), 2 * _CH)
            ebase = pl.multiple_of(ci * _CH, _CH)
            return (pltpu.make_async_copy(sw_hbm.at[pl.ds(sbase, 2 * _CH)],
                                          sw_v, isem),
                    pltpu.make_async_copy(dst_hbm.at[pl.ds(ebase, _CH)],
                                          dst_v, isem))

        def idx_issue(slot, st):
            for cp in idx_copies(slot, st):
                cp.start()

        def gather_issue(slot, st):
            sw_v, dst_v, rows_v, gsem, isem = st
            for cp in idx_copies(slot, st):
                cp.wait()
            if feature_split:
                roff = c * _N

                @pl.loop(0, _CH, step=16)
                def _(i):
                    sw_v[pl.ds(i, 16)] = sw_v[pl.ds(i, 16)] + roff

            pltpu.make_async_copy(sup_hbm.at[sw_v.at[pl.ds(0, _CH)]],
                                  rows_v, gsem).start()

        def process(st):
            sw_v, dst_v, rows_v, gsem, isem = st
            pltpu.make_async_copy(sup_hbm.at[sw_v.at[pl.ds(0, _CH)]],
                                  rows_v, gsem).wait()

            @pl.loop(0, _CH, step=16)
            def _(g):
                wvec = plsc.bitcast(sw_v[pl.ds(_CH + g, 16)], jnp.float32)
                for j in range(16):
                    wj = wvec.at[jnp.full((16,), j, jnp.int32)].get(
                        mode="promise_in_bounds")

                    @pl.loop(0, F2, step=16)
                    def _(f):
                        rows_v[g + j, pl.ds(f, 16)] = (
                            rows_v[g + j, pl.ds(f, 16)] * wj)

            pltpu.sync_copy(rows_v, acc_sh.at[dst_v], add=True)

        @pl.when(in_range(0))
        def _():
            idx_issue(0, sets[0])
            gather_issue(0, sets[0])

        @pl.when(in_range(1))
        def _():
            idx_issue(1, sets[1])

        @pl.loop(0, pairs)
        def _(kk):
            for p in (0, 1):
                slot = 2 * kk + p

                @pl.when(in_range(slot + 1))
                def _():
                    gather_issue(slot + 1, sets[1 - p])

                @pl.when(in_range(slot))
                def _():
                    process(sets[p])

                @pl.when(in_range(slot + 2))
                def _():
                    idx_issue(slot + 2, sets[p])

        plsc.subcore_barrier()
        pltpu.sync_copy(acc_sh.at[pl.ds(base_row, rows_per_sub)],
                        out_hbm.at[c].at[pl.ds(base_row, rows_per_sub)])

        @pl.when(s == _NSUB - 1)
        def _():
            pltpu.sync_copy(acc_sh.at[pl.ds(tail_base, tail_rows)],
                            out_hbm.at[c].at[pl.ds(tail_base, tail_rows)])

    return k(sup, sw, dst)


def kernel(x, edge_index, edge_weight, W1, b1, W2, b2):
    src = edge_index[0]
    dst = edge_index[1]
    n_chunks = _E // _CH
    w_i32 = jax.lax.bitcast_convert_type(edge_weight, jnp.int32)
    sw = jnp.concatenate([src.reshape(n_chunks, _CH),
                          w_i32.reshape(n_chunks, _CH)], axis=1).reshape(-1)
    sup1 = _matmul_halves(x, W1, 1000)
    agg1 = _sc_spmm(sup1.reshape(2 * _N, 128), sw, dst,
                    feature_split=True)
    sup2 = _relu_matmul(agg1, b1, W2, 1000)
    agg2 = _sc_spmm(sup2, sw, dst,
                    feature_split=False)
    logits = _combine_bias(agg2, b2, 1000)
    return (logits, logits)


# speedup vs baseline: 7.8035x; 1.1889x over previous
import dataclasses
import functools

import jax
import jax.numpy as jnp
from jax import lax
from jax.experimental import pallas as pl
from jax.experimental.pallas import tpu as pltpu
from jax.experimental.pallas import tpu_sc as plsc

_N = 10000
_E = 160000
_NSUB = 16
_NCORE = 2
_CH = 128


def _matmul_halves(x, W, Rb):
    N, K = x.shape
    F2 = W.shape[1] // 2

    def body(x_ref, w_ref, o_ref):
        o_ref[0] = jnp.dot(x_ref[...], w_ref[...],
                           preferred_element_type=jnp.float32)

    return pl.pallas_call(
        body,
        grid=(2, N // Rb),
        in_specs=[pl.BlockSpec((Rb, K), lambda h, r: (r, 0)),
                  pl.BlockSpec((K, F2), lambda h, r: (0, h))],
        out_specs=pl.BlockSpec((1, Rb, F2), lambda h, r: (h, r, 0)),
        out_shape=jax.ShapeDtypeStruct((2, N, F2), jnp.float32),
    )(x, W)


def _relu_matmul(agg, b, W, Rb):
    _, N, K2 = agg.shape
    F = W.shape[1]

    def body(a_ref, b_ref, w_ref, o_ref):
        h0 = jnp.maximum(a_ref[0] + b_ref[0], 0.0)
        h1 = jnp.maximum(a_ref[1] + b_ref[1], 0.0)
        w = w_ref[...]
        o_ref[...] = (jnp.dot(h0, w[:K2], preferred_element_type=jnp.float32)
                      + jnp.dot(h1, w[K2:], preferred_element_type=jnp.float32))

    return pl.pallas_call(
        body,
        grid=(N // Rb,),
        in_specs=[pl.BlockSpec((2, Rb, K2), lambda r: (0, r, 0)),
                  pl.BlockSpec((2, 1, K2), lambda r: (0, 0, 0)),
                  pl.BlockSpec((2 * K2, F), lambda r: (0, 0))],
        out_specs=pl.BlockSpec((Rb, F), lambda r: (r, 0)),
        out_shape=jax.ShapeDtypeStruct((N, F), jnp.float32),
    )(agg, b.reshape(2, 1, K2), W)


def _combine_bias(parts, b, Rb):
    _, N, F = parts.shape

    def body(a_ref, b_ref, o_ref):
        o_ref[...] = a_ref[0] + a_ref[1] + b_ref[...]

    return pl.pallas_call(
        body,
        grid=(N // Rb,),
        in_specs=[pl.BlockSpec((2, Rb, F), lambda r: (0, r, 0)),
                  pl.BlockSpec((1, F), lambda r: (0, 0))],
        out_specs=pl.BlockSpec((Rb, F), lambda r: (r, 0)),
        out_shape=jax.ShapeDtypeStruct((N, F), jnp.float32),
    )(parts, b.reshape(1, F))


def _sc_spmm(sup, sw, dst, feature_split):
    n_sup_rows, F2 = sup.shape
    mesh = plsc.VectorSubcoreMesh(core_axis_name="c", subcore_axis_name="s")
    n_chunks = _E // _CH
    chunks_per_core = n_chunks // _NCORE
    rows_per_sub = 624
    tail_base = _NSUB * rows_per_sub
    tail_rows = _N - tail_base

    total = n_chunks if feature_split else chunks_per_core
    max_slots = (total + _NSUB - 1) // _NSUB
    pairs = (max_slots + 1) // 2

    cp = pltpu.CompilerParams()
    if "needs_layout_passes" in pltpu.CompilerParams.__dataclass_fields__:
        cp = dataclasses.replace(cp, needs_layout_passes=False)

    @functools.partial(
        pl.kernel,
        mesh=mesh,
        compiler_params=cp,
        out_type=jax.ShapeDtypeStruct((2, _N, F2), jnp.float32),
        scratch_types=[
            pltpu.VMEM((2 * _CH,), jnp.int32),
            pltpu.VMEM((_CH,), jnp.int32),
            pltpu.VMEM((_CH,), jnp.int32),
            pltpu.VMEM((_CH, F2), jnp.float32),
            pltpu.VMEM((2 * _CH,), jnp.int32),
            pltpu.VMEM((_CH,), jnp.int32),
            pltpu.VMEM((_CH,), jnp.int32),
            pltpu.VMEM((_CH, F2), jnp.float32),
            pltpu.VMEM_SHARED((_N, F2), jnp.float32),
            pltpu.SemaphoreType.DMA,
            pltpu.SemaphoreType.DMA,
            pltpu.SemaphoreType.DMA,
            pltpu.SemaphoreType.DMA,
            pltpu.SemaphoreType.DMA,
            pltpu.SemaphoreType.DMA,
        ],
    )
    def k(sup_hbm, sw_hbm, dst_hbm, out_hbm,
          sw_a, dst_a, dsts_a, rows_a, sw_b, dst_b, dsts_b, rows_b,
          acc_sh, gsem_a, gsem_b, isem_a, isem_b, ssem_a, ssem_b):
        c = lax.axis_index("c")
        s = lax.axis_index("s")
        sets = ((sw_a, dst_a, dsts_a, rows_a, gsem_a, isem_a, ssem_a),
                (sw_b, dst_b, dsts_b, rows_b, gsem_b, isem_b, ssem_b))
        rows_v = rows_a

        @pl.loop(0, _CH)
        def _(r):
            @pl.loop(0, F2, step=16)
            def _(f):
                rows_v[r, pl.ds(f, 16)] = jnp.zeros((16,), jnp.float32)

        base_row = pl.multiple_of(s * rows_per_sub, 8)
        off = 0
        while off < rows_per_sub:
            sz = min(_CH, rows_per_sub - off)
            pltpu.sync_copy(rows_v.at[pl.ds(0, sz)],
                            acc_sh.at[pl.ds(base_row + off, sz)])
            off += sz

        @pl.when(s == _NSUB - 1)
        def _():
            pltpu.sync_copy(rows_v.at[pl.ds(0, tail_rows)],
                            acc_sh.at[pl.ds(tail_base, tail_rows)])

        plsc.subcore_barrier()

        chunk0 = 0 if feature_split else c * chunks_per_core

        def in_range(slot):
            return s + slot * _NSUB < total

        def idx_copies(slot, st):
            sw_v, dst_v, dsts_v, rows_v, gsem, isem, ssem = st
            ci = chunk0 + s + slot * _NSUB
            sbase = pl.multiple_of(ci * (2 * _CH), 2 * _CH)
            ebase = pl.multiple_of(ci * _CH, _CH)
            return (pltpu.make_async_copy(sw_hbm.at[pl.ds(sbase, 2 * _CH)],
                                          sw_v, isem),
                    pltpu.make_async_copy(dst_hbm.at[pl.ds(ebase, _CH)],
                                          dst_v, isem))

        def idx_issue(slot, st):
            for cp in idx_copies(slot, st):
                cp.start()

        def scatter_copy(st):
            sw_v, dst_v, dsts_v, rows_v, gsem, isem, ssem = st
            return pltpu.make_async_copy(rows_v, acc_sh.at[dsts_v], ssem)

        def gather_issue(slot, st, drain):
            sw_v, dst_v, dsts_v, rows_v, gsem, isem, ssem = st

            if drain is not False:
                @pl.when(drain)
                def _():
                    scatter_copy(st).wait()

            for cp in idx_copies(slot, st):
                cp.wait()
            if feature_split:
                roff = c * _N

                @pl.loop(0, _CH, step=16)
                def _(i):
                    sw_v[pl.ds(i, 16)] = sw_v[pl.ds(i, 16)] + roff

            pltpu.make_async_copy(sup_hbm.at[sw_v.at[pl.ds(0, _CH)]],
                                  rows_v, gsem).start()

        def process(st):
            sw_v, dst_v, dsts_v, rows_v, gsem, isem, ssem = st
            pltpu.make_async_copy(sup_hbm.at[sw_v.at[pl.ds(0, _CH)]],
                                  rows_v, gsem).wait()

            @pl.loop(0, _CH, step=16)
            def _(g):
                dsts_v[pl.ds(g, 16)] = dst_v[pl.ds(g, 16)]
                wvec = plsc.bitcast(sw_v[pl.ds(_CH + g, 16)], jnp.float32)
                for j in range(16):
                    wj = wvec.at[jnp.full((16,), j, jnp.int32)].get(
                        mode="promise_in_bounds")

                    @pl.loop(0, F2, step=16)
                    def _(f):
                        rows_v[g + j, pl.ds(f, 16)] = (
                            rows_v[g + j, pl.ds(f, 16)] * wj)

            scatter_copy(st).start()

        @pl.when(in_range(0))
        def _():
            idx_issue(0, sets[0])
            gather_issue(0, sets[0], False)

        @pl.when(in_range(1))
        def _():
            idx_issue(1, sets[1])

        @pl.loop(0, pairs)
        def _(kk):
            for p in (0, 1):
                slot = 2 * kk + p

                @pl.when(in_range(slot + 1))
                def _():
                    gather_issue(slot + 1, sets[1 - p], slot + 1 >= 2)

                @pl.when(in_range(slot))
                def _():
                    process(sets[p])

                @pl.when(in_range(slot + 2))
                def _():
                    idx_issue(slot + 2, sets[p])

        for last in (2 * pairs - 2, 2 * pairs - 1):
            @pl.when(in_range(last))
            def _():
                scatter_copy(sets[last % 2]).wait()

        plsc.subcore_barrier()
        pltpu.sync_copy(acc_sh.at[pl.ds(base_row, rows_per_sub)],
                        out_hbm.at[c].at[pl.ds(base_row, rows_per_sub)])

        @pl.when(s == _NSUB - 1)
        def _():
            pltpu.sync_copy(acc_sh.at[pl.ds(tail_base, tail_rows)],
                            out_hbm.at[c].at[pl.ds(tail_base, tail_rows)])

    return k(sup, sw, dst)


def kernel(x, edge_index, edge_weight, W1, b1, W2, b2):
    src = edge_index[0]
    dst = edge_index[1]
    n_chunks = _E // _CH
    w_i32 = jax.lax.bitcast_convert_type(edge_weight, jnp.int32)
    sw = jnp.concatenate([src.reshape(n_chunks, _CH),
                          w_i32.reshape(n_chunks, _CH)], axis=1).reshape(-1)
    sup1 = _matmul_halves(x, W1, 1000)
    agg1 = _sc_spmm(sup1.reshape(2 * _N, 128), sw, dst,
                    feature_split=True)
    sup2 = _relu_matmul(agg1, b1, W2, 1000)
    agg2 = _sc_spmm(sup2, sw, dst,
                    feature_split=False)
    logits = _combine_bias(agg2, b2, 1000)
    return (logits, logits)
